# baseline (speedup 1.0000x reference)
import jax
import jax.numpy as jnp
from jax.experimental import pallas as pl

_ROWS_PER_BLOCK = 2048


def _body(x_ref, o_ref):
    b = x_ref[...]
    avg = (jnp.roll(b, -3, axis=1) + jnp.roll(b, -6, axis=1)
           + jnp.roll(b, -9, axis=1)) * (1.0 / 3.0)
    cols = jax.lax.broadcasted_iota(jnp.int32, b.shape, 1)
    o_ref[...] = jnp.where(cols < 3, avg, b)


def kernel(x):
    B, F, J, C = x.shape
    rows = B * F
    row = J * C
    x2 = x.reshape(rows, row)
    out = pl.pallas_call(
        _body,
        grid=(rows // _ROWS_PER_BLOCK,),
        in_specs=[pl.BlockSpec((_ROWS_PER_BLOCK, row), lambda i: (i, 0))],
        out_specs=pl.BlockSpec((_ROWS_PER_BLOCK, row), lambda i: (i, 0)),
        out_shape=jax.ShapeDtypeStruct((rows, row), x.dtype),
    )(x2)
    return out.reshape(B, F, J, C)
